# Initial kernel scaffold; baseline (speedup 1.0000x reference)
import jax
import jax.numpy as jnp
from jax.experimental import pallas as pl


def kernel(embeds, is_numbers, Wq, Wk, Wv, Wo):
    raise NotImplementedError("write your pallas kernel here")


# speedup vs baseline: 21.3797x; 21.3797x over previous
import functools

import jax
import jax.numpy as jnp
from jax.experimental import pallas as pl
from jax.experimental.pallas import tpu as pltpu

N_LEFT = 8
N_RIGHT = 8
N_HEADS = 12
_BT = 128


def _attn_body(e_ref, isn_ref, wq_ref, wk_ref, wv_ref, wo_ref, o_ref, *, T, D):
    H = N_HEADS
    dh = D // H
    W = N_LEFT + N_RIGHT
    i = pl.program_id(1)
    t0 = i * _BT

    eh = e_ref[0, pl.ds(t0, _BT + W), :]
    ec = eh[N_LEFT:N_LEFT + _BT, :]

    q = jnp.dot(ec, wq_ref[...], preferred_element_type=jnp.float32)
    k = jnp.dot(eh, wk_ref[...], preferred_element_type=jnp.float32)
    v = jnp.dot(eh, wv_ref[...], preferred_element_type=jnp.float32)

    qi = jax.lax.broadcasted_iota(jnp.int32, (_BT, _BT + W), 0)
    kj = jax.lax.broadcasted_iota(jnp.int32, (_BT, _BT + W), 1)
    pos_k = t0 + kj - N_LEFT
    mask = (kj >= qi) & (kj <= qi + W) & (pos_k >= 0) & (pos_k < T)

    scale = 1.0 / (dh ** 0.5)
    outs = []
    for h in range(H):
        sl = slice(h * dh, (h + 1) * dh)
        s = jax.lax.dot_general(q[:, sl], k[:, sl],
                                (((1,), (1,)), ((), ())),
                                preferred_element_type=jnp.float32)
        s = jnp.where(mask, s * scale, jnp.float32(-1e9))
        m = jnp.max(s, axis=1, keepdims=True)
        p = jnp.exp(s - m)
        p = p / jnp.sum(p, axis=1, keepdims=True)
        outs.append(jnp.dot(p, v[:, sl], preferred_element_type=jnp.float32))
    attn = jnp.concatenate(outs, axis=1)
    a = jnp.dot(attn, wo_ref[...], preferred_element_type=jnp.float32)

    msk = isn_ref[0] != 0
    o_ref[0] = jnp.where(msk, a, ec)


def kernel(embeds, is_numbers, Wq, Wk, Wv, Wo):
    B, T, D = embeds.shape
    W = N_LEFT + N_RIGHT
    e_pad = jnp.pad(embeds, ((0, 0), (N_LEFT, N_RIGHT), (0, 0)))
    isn = is_numbers.astype(jnp.int32).reshape(B, T, 1)
    grid = (B, T // _BT)
    return pl.pallas_call(
        functools.partial(_attn_body, T=T, D=D),
        grid=grid,
        in_specs=[
            pl.BlockSpec((1, T + W, D), lambda b, i: (b, 0, 0)),
            pl.BlockSpec((1, _BT, 1), lambda b, i: (b, i, 0)),
            pl.BlockSpec((D, D), lambda b, i: (0, 0)),
            pl.BlockSpec((D, D), lambda b, i: (0, 0)),
            pl.BlockSpec((D, D), lambda b, i: (0, 0)),
            pl.BlockSpec((D, D), lambda b, i: (0, 0)),
        ],
        out_specs=pl.BlockSpec((1, _BT, D), lambda b, i: (b, i, 0)),
        out_shape=jax.ShapeDtypeStruct((B, T, D), jnp.float32),
        compiler_params=pltpu.CompilerParams(
            dimension_semantics=("parallel", "arbitrary"),
        ),
    )(e_pad, isn, Wq, Wk, Wv, Wo)
